# Initial kernel scaffold; baseline (speedup 1.0000x reference)
import jax
import jax.numpy as jnp
from jax.experimental import pallas as pl


def kernel(x, A, B, d):
    raise NotImplementedError("write your pallas kernel here")


# speedup vs baseline: 1.1173x; 1.1173x over previous
import functools

import jax
import jax.numpy as jnp
from jax.experimental import pallas as pl

_R = 64
_K = 8
_SCALE = 2.0


def _body(x_ref, at_ref, bt_ref, d_ref, out_ref):
    x = x_ref[...]
    y = jax.lax.dot_general(
        x, at_ref[...], (((1,), (0,)), ((), ())),
        preferred_element_type=jnp.float32)
    yb = y + d_ref[...]
    a = jnp.abs(yb)

    idx = jax.lax.broadcasted_iota(jnp.int32, a.shape, 1)
    mask = jnp.zeros(a.shape, dtype=jnp.bool_)
    work = a
    for _ in range(_K):
        m = jnp.max(work, axis=1, keepdims=True)
        is_max = work == m
        fi = jnp.min(jnp.where(is_max, idx, _R), axis=1, keepdims=True)
        sel = idx == fi
        mask = jnp.logical_or(mask, sel)
        work = jnp.where(sel, -jnp.inf, work)

    act = jnp.where(mask, y, 0.0)
    out_ref[...] = jax.lax.dot_general(
        act, bt_ref[...], (((1,), (0,)), ((), ())),
        preferred_element_type=jnp.float32) * _SCALE


@jax.jit
def kernel(x, A, B, d):
    n, in_f = x.shape
    out_f = B.shape[0]
    bt = 512
    grid = (n // bt,)
    return pl.pallas_call(
        _body,
        grid=grid,
        in_specs=[
            pl.BlockSpec((bt, in_f), lambda i: (i, 0)),
            pl.BlockSpec((in_f, _R), lambda i: (0, 0)),
            pl.BlockSpec((_R, out_f), lambda i: (0, 0)),
            pl.BlockSpec((1, _R), lambda i: (0, 0)),
        ],
        out_specs=pl.BlockSpec((bt, out_f), lambda i: (i, 0)),
        out_shape=jax.ShapeDtypeStruct((n, out_f), jnp.float32),
    )(x, A.T, B.T, d.reshape(1, _R))


# speedup vs baseline: 1.2397x; 1.1095x over previous
import functools

import jax
import jax.numpy as jnp
from jax.experimental import pallas as pl
from jax.experimental.pallas import tpu as pltpu

_R = 64
_K = 8
_SCALE = 2.0


def _body(x_ref, at_ref, bt_ref, d_ref, out_ref):
    x = x_ref[...]
    y = jax.lax.dot_general(
        x, at_ref[...], (((1,), (0,)), ((), ())),
        preferred_element_type=jnp.float32)
    yb = y + d_ref[...]
    a = jnp.abs(yb)

    idx = jax.lax.broadcasted_iota(jnp.int32, a.shape, 1)
    mask = jnp.zeros(a.shape, dtype=jnp.bool_)
    work = a
    for _ in range(_K):
        am = jnp.argmax(work, axis=1)
        sel = idx == am[:, None]
        mask = jnp.logical_or(mask, sel)
        work = jnp.where(sel, -jnp.inf, work)

    act = jnp.where(mask, y, 0.0).astype(jnp.bfloat16)
    out_ref[...] = jax.lax.dot_general(
        act, bt_ref[...], (((1,), (0,)), ((), ())),
        preferred_element_type=jnp.float32) * _SCALE


@jax.jit
def kernel(x, A, B, d):
    n, in_f = x.shape
    out_f = B.shape[0]
    bt = 512
    grid = (n // bt,)
    return pl.pallas_call(
        _body,
        grid=grid,
        in_specs=[
            pl.BlockSpec((bt, in_f), lambda i: (i, 0)),
            pl.BlockSpec((in_f, _R), lambda i: (0, 0)),
            pl.BlockSpec((_R, out_f), lambda i: (0, 0)),
            pl.BlockSpec((1, _R), lambda i: (0, 0)),
        ],
        out_specs=pl.BlockSpec((bt, out_f), lambda i: (i, 0)),
        out_shape=jax.ShapeDtypeStruct((n, out_f), jnp.float32),
        compiler_params=pltpu.CompilerParams(
            dimension_semantics=("parallel",)),
    )(x, A.T, B.T.astype(jnp.bfloat16), d.reshape(1, _R))


# speedup vs baseline: 1.2430x; 1.0027x over previous
import functools

import jax
import jax.numpy as jnp
from jax.experimental import pallas as pl
from jax.experimental.pallas import tpu as pltpu

_R = 64
_K = 8
_SCALE = 2.0


def _body(x_ref, at_ref, bt_ref, d_ref, out_ref):
    x = x_ref[...].astype(jnp.bfloat16)
    y = jax.lax.dot_general(
        x, at_ref[...], (((1,), (0,)), ((), ())),
        preferred_element_type=jnp.float32)
    yb = y + d_ref[...]
    a = jnp.abs(yb)

    idx = jax.lax.broadcasted_iota(jnp.int32, a.shape, 1)
    mask = jnp.zeros(a.shape, dtype=jnp.bool_)
    work = a
    for _ in range(_K):
        am = jnp.argmax(work, axis=1)
        sel = idx == am[:, None]
        mask = jnp.logical_or(mask, sel)
        work = jnp.where(sel, -jnp.inf, work)

    act = jnp.where(mask, y, 0.0).astype(jnp.bfloat16)
    out_ref[...] = jax.lax.dot_general(
        act, bt_ref[...], (((1,), (0,)), ((), ())),
        preferred_element_type=jnp.float32) * _SCALE


@jax.jit
def kernel(x, A, B, d):
    n, in_f = x.shape
    out_f = B.shape[0]
    bt = 512
    grid = (n // bt,)
    return pl.pallas_call(
        _body,
        grid=grid,
        in_specs=[
            pl.BlockSpec((bt, in_f), lambda i: (i, 0)),
            pl.BlockSpec((in_f, _R), lambda i: (0, 0)),
            pl.BlockSpec((_R, out_f), lambda i: (0, 0)),
            pl.BlockSpec((1, _R), lambda i: (0, 0)),
        ],
        out_specs=pl.BlockSpec((bt, out_f), lambda i: (i, 0)),
        out_shape=jax.ShapeDtypeStruct((n, out_f), jnp.float32),
        compiler_params=pltpu.CompilerParams(
            dimension_semantics=("parallel",)),
    )(x, A.T.astype(jnp.bfloat16), B.T.astype(jnp.bfloat16), d.reshape(1, _R))


# speedup vs baseline: 1.2458x; 1.0022x over previous
import functools

import jax
import jax.numpy as jnp
from jax.experimental import pallas as pl
from jax.experimental.pallas import tpu as pltpu

_R = 64
_K = 8
_SCALE = 2.0


def _body(x_ref, at_ref, bt_ref, d_ref, out_ref):
    x = x_ref[...].astype(jnp.bfloat16)
    y = jax.lax.dot_general(
        x, at_ref[...], (((1,), (0,)), ((), ())),
        preferred_element_type=jnp.float32)
    yb = y + d_ref[...]
    a = jnp.abs(yb)

    idx = jax.lax.broadcasted_iota(jnp.int32, a.shape, 1)
    mask = jnp.zeros(a.shape, dtype=jnp.bool_)
    work = a
    for _ in range(_K):
        am = jnp.argmax(work, axis=1)
        sel = idx == am[:, None]
        mask = jnp.logical_or(mask, sel)
        work = jnp.where(sel, -jnp.inf, work)

    act = jnp.where(mask, y + y, 0.0).astype(jnp.bfloat16)
    out_ref[...] = jax.lax.dot_general(
        act, bt_ref[...], (((1,), (0,)), ((), ())),
        preferred_element_type=jnp.float32)


@jax.jit
def kernel(x, A, B, d):
    n, in_f = x.shape
    out_f = B.shape[0]
    bt = 512
    grid = (n // bt,)
    return pl.pallas_call(
        _body,
        grid=grid,
        in_specs=[
            pl.BlockSpec((bt, in_f), lambda i: (i, 0)),
            pl.BlockSpec((in_f, _R), lambda i: (0, 0)),
            pl.BlockSpec((_R, out_f), lambda i: (0, 0)),
            pl.BlockSpec((1, _R), lambda i: (0, 0)),
        ],
        out_specs=pl.BlockSpec((bt, out_f), lambda i: (i, 0)),
        out_shape=jax.ShapeDtypeStruct((n, out_f), jnp.float32),
        compiler_params=pltpu.CompilerParams(
            dimension_semantics=("parallel",)),
    )(x, A.T.astype(jnp.bfloat16), B.T.astype(jnp.bfloat16), d.reshape(1, _R))
